# baseline (speedup 1.0000x reference)
import functools

import jax
import jax.numpy as jnp
from jax import lax
from jax.experimental import pallas as pl
from jax.experimental.pallas import tpu as pltpu
from jax.experimental.pallas import tpu_sc as plsc

CHUNKS_PER_ROW = 16
CHUNK_D = 49152 // CHUNKS_PER_ROW
NUM_WORKERS = 32
ROWS_PER_DMA = 16

_mesh = plsc.VectorSubcoreMesh(core_axis_name="c", subcore_axis_name="s")


@functools.partial(
    pl.kernel,
    out_type=jax.ShapeDtypeStruct((8192, CHUNK_D), jnp.float32),
    mesh=_mesh,
    scratch_types=[
        pltpu.VMEM((256,), jnp.int32),
        pltpu.VMEM((ROWS_PER_DMA, CHUNK_D), jnp.float32),
        pltpu.VMEM((ROWS_PER_DMA, CHUNK_D), jnp.float32),
        pltpu.SemaphoreType.DMA,
        pltpu.SemaphoreType.DMA,
    ],
)
def _gather_rows(idx_hbm, table_hbm, out_hbm, idx_v, buf0, buf1, sem0, sem1):
    wid = lax.axis_index("s") * 2 + lax.axis_index("c")
    base = wid * 256
    pltpu.sync_copy(idx_hbm.at[pl.ds(base, 256)], idx_v)

    bufs = (buf0, buf1)
    sems = (sem0, sem1)
    copies = [None, None]
    n_iters = 256 // ROWS_PER_DMA
    copies[0] = pltpu.async_copy(
        table_hbm.at[idx_v.at[pl.ds(0, ROWS_PER_DMA)]], buf0, sem0)
    for t in range(n_iters):
        nt = t + 1
        if nt < n_iters:
            copies[nt % 2] = pltpu.async_copy(
                table_hbm.at[idx_v.at[pl.ds(nt * ROWS_PER_DMA, ROWS_PER_DMA)]],
                bufs[nt % 2], sems[nt % 2])
        copies[t % 2].wait()
        pltpu.sync_copy(
            bufs[t % 2], out_hbm.at[pl.ds(base + t * ROWS_PER_DMA, ROWS_PER_DMA)])


def kernel(prefix_tokens, table):
    b, s = prefix_tokens.shape
    emb_dim = table.shape[1]
    table_chunks = table.reshape(table.shape[0] * CHUNKS_PER_ROW, CHUNK_D)
    idx = prefix_tokens.reshape(-1).astype(jnp.int32)
    chunk_idx = (idx[:, None] * CHUNKS_PER_ROW
                 + jnp.arange(CHUNKS_PER_ROW, dtype=jnp.int32)[None, :]).reshape(-1)
    out = _gather_rows(chunk_idx, table_chunks)
    return out.reshape(b, s, emb_dim)
